# baseline (speedup 1.0000x reference)
import functools

import jax
import jax.numpy as jnp
from jax.experimental import pallas as pl
from jax.experimental.pallas import tpu as pltpu

B, D, T, K = 16, 256, 1024, 1024
COMMITMENT_COST = 0.25


def _vq_body(z_ref, w_ref, wt_ref, zsq_ref, wsq_ref, codes_ref, zq_ref, loss_ref):
    z = z_ref[0]
    w = w_ref[...]
    wt = wt_ref[...]
    zsq = zsq_ref[0]
    wsq = wsq_ref[...]
    m = jax.lax.dot_general(w, z, (((1,), (0,)), ((), ())),
                            preferred_element_type=jnp.float32)
    dist = (zsq - 2.0 * m) + wsq
    mn = jnp.min(dist, axis=0, keepdims=True)
    iota_k = jax.lax.broadcasted_iota(jnp.int32, (K, T), 0)
    codes = jnp.min(jnp.where(dist == mn, iota_k, K), axis=0).astype(jnp.int32)
    codes_ref[0, 0, :] = codes
    onehot = (jax.lax.broadcasted_iota(jnp.int32, (K, T), 0)
              == codes[None, :]).astype(jnp.float32)
    zq = jax.lax.dot_general(wt, onehot, (((1,), (0,)), ((), ())),
                             preferred_element_type=jnp.float32,
                             precision=jax.lax.Precision.HIGHEST)
    zq_ref[0] = zq
    diff = zq - z
    loss_ref[0, 0, :] = jnp.full((128,), jnp.sum(diff * diff), jnp.float32)


@functools.partial(jax.jit, static_argnames=())
def kernel(z, W):
    Wt = W.T
    z_flat = jnp.transpose(z, (0, 2, 1)).reshape(-1, D)
    zsq = jnp.sum(z_flat ** 2, axis=1).reshape(B, 1, T)
    wsq = jnp.sum(W ** 2, axis=1).reshape(K, 1)
    codes3, zq, loss_parts = pl.pallas_call(
        _vq_body,
        grid=(B,),
        in_specs=[
            pl.BlockSpec((1, D, T), lambda b: (b, 0, 0)),
            pl.BlockSpec((K, D), lambda b: (0, 0)),
            pl.BlockSpec((D, K), lambda b: (0, 0)),
            pl.BlockSpec((1, 1, T), lambda b: (b, 0, 0)),
            pl.BlockSpec((K, 1), lambda b: (0, 0)),
        ],
        out_specs=[
            pl.BlockSpec((1, 1, T), lambda b: (b, 0, 0)),
            pl.BlockSpec((1, D, T), lambda b: (b, 0, 0)),
            pl.BlockSpec((1, 1, 128), lambda b: (b, 0, 0)),
        ],
        out_shape=[
            jax.ShapeDtypeStruct((B, 1, T), jnp.int32),
            jax.ShapeDtypeStruct((B, D, T), jnp.float32),
            jax.ShapeDtypeStruct((B, 1, 128), jnp.float32),
        ],
        compiler_params=pltpu.CompilerParams(
            dimension_semantics=("parallel",),
        ),
    )(z, W, Wt, zsq, wsq)
    codes = codes3.reshape(B * T)
    sq_err_sum = jnp.sum(loss_parts[:, 0, 0])
    vq_loss = (1.0 + COMMITMENT_COST) * sq_err_sum / (B * D * T)
    return zq, vq_loss, codes


# speedup vs baseline: 1.7875x; 1.7875x over previous
import functools

import jax
import jax.numpy as jnp
from jax.experimental import pallas as pl
from jax.experimental.pallas import tpu as pltpu

B, D, T, K = 16, 256, 1024, 1024
COMMITMENT_COST = 0.25


def _vq_body(z_ref, w_ref, wt_ref, zsq_ref, wsq_ref, codes_ref, zq_ref, loss_ref):
    z = z_ref[0]
    w = w_ref[...]
    wt = wt_ref[...]
    zsq = zsq_ref[0]
    wsq = wsq_ref[...]
    m = jax.lax.dot_general(w, z, (((1,), (0,)), ((), ())),
                            preferred_element_type=jnp.float32)
    dist = (zsq - 2.0 * m) + wsq
    mn = jnp.min(dist, axis=0, keepdims=True)
    iota_k = jax.lax.broadcasted_iota(jnp.int32, (K, T), 0)
    codes = jnp.min(jnp.where(dist == mn, iota_k, K), axis=0).astype(jnp.int32)
    codes_ref[0, 0, :] = codes
    onehot = (jax.lax.broadcasted_iota(jnp.int32, (K, T), 0)
              == codes[None, :]).astype(jnp.float32)
    zq = jax.lax.dot_general(wt, onehot, (((1,), (0,)), ((), ())),
                             preferred_element_type=jnp.float32)
    zq_ref[0] = zq
    diff = zq - z
    loss_ref[0, 0, :] = jnp.full((128,), jnp.sum(diff * diff), jnp.float32)


@functools.partial(jax.jit, static_argnames=())
def kernel(z, W):
    Wt = W.T
    z_flat = jnp.transpose(z, (0, 2, 1)).reshape(-1, D)
    zsq = jnp.sum(z_flat ** 2, axis=1).reshape(B, 1, T)
    wsq = jnp.sum(W ** 2, axis=1).reshape(K, 1)
    codes3, zq, loss_parts = pl.pallas_call(
        _vq_body,
        grid=(B,),
        in_specs=[
            pl.BlockSpec((1, D, T), lambda b: (b, 0, 0)),
            pl.BlockSpec((K, D), lambda b: (0, 0)),
            pl.BlockSpec((D, K), lambda b: (0, 0)),
            pl.BlockSpec((1, 1, T), lambda b: (b, 0, 0)),
            pl.BlockSpec((K, 1), lambda b: (0, 0)),
        ],
        out_specs=[
            pl.BlockSpec((1, 1, T), lambda b: (b, 0, 0)),
            pl.BlockSpec((1, D, T), lambda b: (b, 0, 0)),
            pl.BlockSpec((1, 1, 128), lambda b: (b, 0, 0)),
        ],
        out_shape=[
            jax.ShapeDtypeStruct((B, 1, T), jnp.int32),
            jax.ShapeDtypeStruct((B, D, T), jnp.float32),
            jax.ShapeDtypeStruct((B, 1, 128), jnp.float32),
        ],
        compiler_params=pltpu.CompilerParams(
            dimension_semantics=("parallel",),
        ),
    )(z, W, Wt, zsq, wsq)
    codes = codes3.reshape(B * T)
    sq_err_sum = jnp.sum(loss_parts[:, 0, 0])
    vq_loss = (1.0 + COMMITMENT_COST) * sq_err_sum / (B * D * T)
    return zq, vq_loss, codes


# speedup vs baseline: 1.9837x; 1.1098x over previous
import functools

import jax
import jax.numpy as jnp
from jax.experimental import pallas as pl
from jax.experimental.pallas import tpu as pltpu

B, D, T, K = 16, 256, 1024, 1024
R = 128
COMMITMENT_COST = 0.25


def _vq_body(z_ref, w_ref, wtb_ref, zsq_ref, wsq_ref, codes_ref, zq_ref,
             loss_ref):
    z = z_ref[0]
    w = w_ref[...]
    wtb = wtb_ref[...]
    zsq = zsq_ref[0]
    wsq = wsq_ref[...]
    m = jax.lax.dot_general(w, z, (((1,), (0,)), ((), ())),
                            preferred_element_type=jnp.float32)

    iota_r = jax.lax.broadcasted_iota(jnp.int32, (R, T), 0).astype(jnp.float32)
    val = (zsq - 2.0 * m[0:R, :]) + wsq[0:R, :]
    idx = iota_r
    for i in range(1, K // R):
        d = (zsq - 2.0 * m[i * R:(i + 1) * R, :]) + wsq[i * R:(i + 1) * R, :]
        take = d < val
        val = jnp.where(take, d, val)
        idx = jnp.where(take, iota_r + jnp.float32(i * R), idx)
    s = R // 2
    while s >= 1:
        av, bv = val[:s], val[s:2 * s]
        ai, bi = idx[:s], idx[s:2 * s]
        take = (bv < av) | ((bv == av) & (bi < ai))
        val = jnp.where(take, bv, av)
        idx = jnp.where(take, bi, ai)
        s //= 2
    codes = idx.astype(jnp.int32)
    codes_ref[0] = codes
    onehot = (jax.lax.broadcasted_iota(jnp.int32, (K, T), 0)
              == codes).astype(jnp.bfloat16)
    zq = jax.lax.dot_general(wtb, onehot, (((1,), (0,)), ((), ())),
                             preferred_element_type=jnp.float32)
    zq_ref[0] = zq
    loss_ref[0, 0, :] = jnp.full((128,), jnp.sum(val), jnp.float32)


@functools.partial(jax.jit, static_argnames=())
def kernel(z, W):
    Wtb = W.T.astype(jnp.bfloat16)
    z_flat = jnp.transpose(z, (0, 2, 1)).reshape(-1, D)
    zsq = jnp.sum(z_flat ** 2, axis=1).reshape(B, 1, T)
    wsq = jnp.sum(W ** 2, axis=1).reshape(K, 1)
    codes3, zq, loss_parts = pl.pallas_call(
        _vq_body,
        grid=(B,),
        in_specs=[
            pl.BlockSpec((1, D, T), lambda b: (b, 0, 0)),
            pl.BlockSpec((K, D), lambda b: (0, 0)),
            pl.BlockSpec((D, K), lambda b: (0, 0)),
            pl.BlockSpec((1, 1, T), lambda b: (b, 0, 0)),
            pl.BlockSpec((K, 1), lambda b: (0, 0)),
        ],
        out_specs=[
            pl.BlockSpec((1, 1, T), lambda b: (b, 0, 0)),
            pl.BlockSpec((1, D, T), lambda b: (b, 0, 0)),
            pl.BlockSpec((1, 1, 128), lambda b: (b, 0, 0)),
        ],
        out_shape=[
            jax.ShapeDtypeStruct((B, 1, T), jnp.int32),
            jax.ShapeDtypeStruct((B, D, T), jnp.float32),
            jax.ShapeDtypeStruct((B, 1, 128), jnp.float32),
        ],
        compiler_params=pltpu.CompilerParams(
            dimension_semantics=("parallel",),
        ),
    )(z, W, Wtb, zsq, wsq)
    codes = codes3.reshape(B * T)
    sq_err_sum = jnp.sum(loss_parts[:, 0, 0])
    vq_loss = (1.0 + COMMITMENT_COST) * sq_err_sum / (B * D * T)
    return zq, vq_loss, codes


# speedup vs baseline: 2.4070x; 1.2134x over previous
import functools

import jax
import jax.numpy as jnp
from jax.experimental import pallas as pl
from jax.experimental.pallas import tpu as pltpu

B, D, T, K = 16, 256, 1024, 1024
R = 128
COMMITMENT_COST = 0.25


def _vq_body(z_ref, w_ref, wb_ref, codes_ref, zq_ref, loss_ref):
    z = z_ref[0]
    w = w_ref[...]
    wb = wb_ref[...]
    zsq = jnp.sum(z * z, axis=0, keepdims=True)
    wsq = jnp.sum(w * w, axis=1, keepdims=True)
    m = jax.lax.dot_general(w, z, (((1,), (0,)), ((), ())),
                            preferred_element_type=jnp.float32)

    iota_r = jax.lax.broadcasted_iota(jnp.int32, (R, T), 0).astype(jnp.float32)
    val = (zsq - 2.0 * m[0:R, :]) + wsq[0:R, :]
    idx = iota_r
    for i in range(1, K // R):
        d = (zsq - 2.0 * m[i * R:(i + 1) * R, :]) + wsq[i * R:(i + 1) * R, :]
        take = d < val
        val = jnp.where(take, d, val)
        idx = jnp.where(take, iota_r + jnp.float32(i * R), idx)
    s = R // 2
    while s >= 1:
        av, bv = val[:s], val[s:2 * s]
        ai, bi = idx[:s], idx[s:2 * s]
        take = (bv < av) | ((bv == av) & (bi < ai))
        val = jnp.where(take, bv, av)
        idx = jnp.where(take, bi, ai)
        s //= 2
    codes = idx.astype(jnp.int32)
    codes_ref[0] = codes
    onehot = (jax.lax.broadcasted_iota(jnp.int32, (K, T), 0)
              == codes).astype(jnp.bfloat16)
    zq = jax.lax.dot_general(wb, onehot, (((0,), (0,)), ((), ())),
                             preferred_element_type=jnp.float32)
    zq_ref[0] = zq
    loss_ref[0, 0, :] = jnp.full((128,), jnp.sum(val), jnp.float32)


@functools.partial(jax.jit, static_argnames=())
def kernel(z, W):
    Wb = W.astype(jnp.bfloat16)
    codes3, zq, loss_parts = pl.pallas_call(
        _vq_body,
        grid=(B,),
        in_specs=[
            pl.BlockSpec((1, D, T), lambda b: (b, 0, 0)),
            pl.BlockSpec((K, D), lambda b: (0, 0)),
            pl.BlockSpec((K, D), lambda b: (0, 0)),
        ],
        out_specs=[
            pl.BlockSpec((1, 1, T), lambda b: (b, 0, 0)),
            pl.BlockSpec((1, D, T), lambda b: (b, 0, 0)),
            pl.BlockSpec((1, 1, 128), lambda b: (b, 0, 0)),
        ],
        out_shape=[
            jax.ShapeDtypeStruct((B, 1, T), jnp.int32),
            jax.ShapeDtypeStruct((B, D, T), jnp.float32),
            jax.ShapeDtypeStruct((B, 1, 128), jnp.float32),
        ],
        compiler_params=pltpu.CompilerParams(
            dimension_semantics=("parallel",),
        ),
    )(z, W, Wb)
    codes = codes3.reshape(B * T)
    sq_err_sum = jnp.sum(loss_parts[:, 0, 0])
    vq_loss = (1.0 + COMMITMENT_COST) * sq_err_sum / (B * D * T)
    return zq, vq_loss, codes
